# Initial kernel scaffold; baseline (speedup 1.0000x reference)
import jax
import jax.numpy as jnp
from jax.experimental import pallas as pl


def kernel(user, item, emb_user_mlp, emb_item_mlp, emb_user_ac, emb_item_ac, W1, b1, W2, b2, Wo, bo):
    raise NotImplementedError("write your pallas kernel here")


# speedup vs baseline: 1.3604x; 1.3604x over previous
import functools

import jax
import jax.numpy as jnp
from jax import lax
from jax.experimental import pallas as pl
from jax.experimental.pallas import tpu as pltpu
from jax.experimental.pallas import tpu_sc as plsc

B = 16384
D = 64
NC, NS = 2, 16
NW = NC * NS
BPW = B // NW
CHUNK = 128
NCH = BPW // CHUNK
H1 = 128
H2 = 64


def _sc_gather(idx3, um, im, ua, ia):
    mesh = plsc.VectorSubcoreMesh(core_axis_name="c", subcore_axis_name="s")

    @functools.partial(
        pl.kernel,
        out_type=jax.ShapeDtypeStruct((6, B, D), jnp.float32),
        mesh=mesh,
        scratch_types=[
            pltpu.VMEM((6, NCH, CHUNK), jnp.int32),
            pltpu.VMEM((2, CHUNK, D), jnp.float32),
            pltpu.SemaphoreType.DMA,
            pltpu.SemaphoreType.DMA,
            pltpu.SemaphoreType.DMA,
        ],
        compiler_params=pltpu.CompilerParams(use_tc_tiling_on_sc=False),
    )
    def k(idx_hbm, um_hbm, im_hbm, ua_hbm, ia_hbm, out_hbm, idx_v, buf, gsem,
          ssem0, ssem1):
        wid = lax.axis_index("s") * NC + lax.axis_index("c")
        gbase = wid * NCH
        pltpu.sync_copy(idx_hbm.at[pl.ds(0, 6), pl.ds(gbase, NCH)], idx_v)

        tables = [um_hbm, im_hbm, ua_hbm, ua_hbm, ia_hbm, ia_hbm]
        steps = [(tables[s], s, c) for s in range(6) for c in range(NCH)]
        ssems = [ssem0, ssem1]
        store = [None, None]
        gath = [None, None]
        for n in range(len(steps) + 1):
            if n < len(steps):
                bsel = n & 1
                if store[bsel] is not None:
                    store[bsel].wait()
                tbl, s, c = steps[n]
                gath[bsel] = pltpu.async_copy(
                    tbl.at[idx_v.at[s, c]], buf.at[bsel], gsem)
            if n >= 1:
                pb = (n - 1) & 1
                gath[pb].wait()
                _, s, c = steps[n - 1]
                store[pb] = pltpu.async_copy(
                    buf.at[pb],
                    out_hbm.at[s, pl.ds(wid * BPW + c * CHUNK, CHUNK)],
                    ssems[pb])
        for bsel in range(2):
            if store[bsel] is not None:
                store[bsel].wait()

    return k(idx3, um, im, ua, ia)


def _tc_body(wos_ref, bo_ref, g_ref, w1_ref, b1_ref, w2_ref, b2_ref, woh_ref,
             out_ref):
    u = g_ref[0]
    it = g_ref[1]
    h = jnp.dot(u, w1_ref[:D, :], preferred_element_type=jnp.float32)
    h = h + jnp.dot(it, w1_ref[D:, :], preferred_element_type=jnp.float32)
    h = jnp.maximum(h + b1_ref[...], 0.0)
    h = jnp.dot(h, w2_ref[...], preferred_element_type=jnp.float32)
    h = jnp.maximum(h + b2_ref[...], 0.0)

    def cos(x, y):
        num = jnp.sum(x * y, axis=1)
        na = jnp.sqrt(jnp.sum(x * x, axis=1))
        nb = jnp.sqrt(jnp.sum(y * y, axis=1))
        return num / jnp.maximum(na * nb, 1e-8)

    s0 = cos(g_ref[2], g_ref[4])
    s1 = cos(g_ref[3], g_ref[5])
    logit = jnp.sum(h * woh_ref[...], axis=1)
    out_ref[...] = logit + s0 * wos_ref[0] + s1 * wos_ref[1] + bo_ref[0]


def _tc_mlp(g, w1, b1, w2, b2, wos, woh, bo, interpret=False):
    bk = 2048
    return pl.pallas_call(
        _tc_body,
        grid=(B // bk,),
        in_specs=[
            pl.BlockSpec(memory_space=pltpu.SMEM),
            pl.BlockSpec(memory_space=pltpu.SMEM),
            pl.BlockSpec((6, bk, D), lambda i: (0, i, 0)),
            pl.BlockSpec((H1, H1), lambda i: (0, 0)),
            pl.BlockSpec((1, H1), lambda i: (0, 0)),
            pl.BlockSpec((H1, H2), lambda i: (0, 0)),
            pl.BlockSpec((1, H2), lambda i: (0, 0)),
            pl.BlockSpec((1, H2), lambda i: (0, 0)),
        ],
        out_specs=pl.BlockSpec((bk,), lambda i: (i,)),
        out_shape=jax.ShapeDtypeStruct((B,), jnp.float32),
        interpret=interpret,
    )(wos, bo, g, w1, b1, w2, b2, woh)


def kernel(user, item, emb_user_mlp, emb_item_mlp, emb_user_ac, emb_item_ac,
           W1, b1, W2, b2, Wo, bo):
    idx3 = jnp.stack([
        user[:, 0], item[:, 0],
        user[:, 1], user[:, 2],
        item[:, 1], item[:, 2],
    ]).reshape(6, B // CHUNK, CHUNK)
    g = _sc_gather(idx3, emb_user_mlp, emb_item_mlp, emb_user_ac, emb_item_ac)
    logit = _tc_mlp(g, W1, b1.reshape(1, H1), W2, b2.reshape(1, H2),
                    Wo[:2, 0], Wo[2:, 0].reshape(1, H2), bo)
    return logit.reshape(B, 1)
